# baseline (speedup 1.0000x reference)
import jax
import jax.numpy as jnp
from jax import lax
from jax.experimental import pallas as pl
from jax.experimental.pallas import tpu as pltpu
from jax.experimental.pallas import tpu_sc as plsc

NC, NS = 2, 16
NW = NC * NS
B = 16384
D = 128
CHUNK = 128
CPW = B // (NW * CHUNK)


def _body(labi, subi, labt, subt, out, idxL, idxS, bufL, bufS,
          gsemL, gsemS, wsemL, wsemS):
    wid = lax.axis_index("s") * NC + lax.axis_index("c")
    row0 = wid * CPW
    pltpu.sync_copy(labi.at[pl.ds(row0, CPW)], idxL)
    pltpu.sync_copy(subi.at[pl.ds(row0, CPW)], idxS)
    gl = [None] * CPW
    gs = [None] * CPW
    wl = [None] * CPW
    ws = [None] * CPW
    gl[0] = pltpu.async_copy(labt.at[idxL.at[0]], bufL.at[0], gsemL)
    gs[0] = pltpu.async_copy(subt.at[idxS.at[0]], bufS.at[0], gsemS)
    for j in range(CPW):
        gl[j].wait()
        gs[j].wait()
        r = (row0 + j) * CHUNK
        wl[j] = pltpu.async_copy(bufL.at[j % 2], out.at[pl.ds(r, CHUNK), 0],
                                 wsemL)
        ws[j] = pltpu.async_copy(bufS.at[j % 2], out.at[pl.ds(r, CHUNK), 1],
                                 wsemS)
        if j + 1 < CPW:
            if j >= 1:
                wl[j - 1].wait()
                ws[j - 1].wait()
            gl[j + 1] = pltpu.async_copy(labt.at[idxL.at[j + 1]],
                                         bufL.at[(j + 1) % 2], gsemL)
            gs[j + 1] = pltpu.async_copy(subt.at[idxS.at[j + 1]],
                                         bufS.at[(j + 1) % 2], gsemS)
    wl[CPW - 2].wait()
    ws[CPW - 2].wait()
    wl[CPW - 1].wait()
    ws[CPW - 1].wait()


def kernel(lab_ids, subject_ids, lab_table, subject_table):
    labi = lab_ids.astype(jnp.int32).reshape(B // CHUNK, CHUNK)
    subi = subject_ids.astype(jnp.int32).reshape(B // CHUNK, CHUNK)
    mesh = plsc.VectorSubcoreMesh(core_axis_name="c", subcore_axis_name="s")
    f = pl.kernel(
        _body,
        mesh=mesh,
        out_type=jax.ShapeDtypeStruct((B, 2, D), jnp.float32),
        scratch_types=[
            pltpu.VMEM((CPW, CHUNK), jnp.int32),
            pltpu.VMEM((CPW, CHUNK), jnp.int32),
            pltpu.VMEM((2, CHUNK, D), jnp.float32),
            pltpu.VMEM((2, CHUNK, D), jnp.float32),
            pltpu.SemaphoreType.DMA,
            pltpu.SemaphoreType.DMA,
            pltpu.SemaphoreType.DMA,
            pltpu.SemaphoreType.DMA,
        ],
    )
    out = f(labi, subi, lab_table, subject_table)
    return out.reshape(B, 2 * D)


# speedup vs baseline: 1.2220x; 1.2220x over previous
import jax
import jax.numpy as jnp
from jax import lax
from jax.experimental import pallas as pl
from jax.experimental.pallas import tpu as pltpu
from jax.experimental.pallas import tpu_sc as plsc

NC, NS = 2, 16
NW = NC * NS
B = 16384
D = 128
CHUNK = 128
CPW = B // (NW * CHUNK)


def _body(labi, subi, labt, subt, out, idxL, idxS, bufL, bufS,
          gsemL, gsemS, wsemL, wsemS):
    wid = lax.axis_index("s") * NC + lax.axis_index("c")
    row0 = wid * CPW
    pltpu.sync_copy(labi.at[pl.ds(row0, CPW)], idxL)
    pltpu.sync_copy(subi.at[pl.ds(row0, CPW)], idxS)
    gl = [None] * CPW
    gs = [None] * CPW
    wl = [None] * CPW
    ws = [None] * CPW
    gl[0] = pltpu.async_copy(labt.at[idxL.at[0]], bufL.at[0], gsemL)
    gs[0] = pltpu.async_copy(subt.at[idxS.at[0]], bufS.at[0], gsemS)
    for j in range(CPW):
        gl[j].wait()
        gs[j].wait()
        r = (row0 + j) * CHUNK
        wl[j] = pltpu.async_copy(bufL.at[j % 2],
                                 out.at[pl.ds(r, CHUNK), pl.ds(0, D)], wsemL)
        ws[j] = pltpu.async_copy(bufS.at[j % 2],
                                 out.at[pl.ds(r, CHUNK), pl.ds(D, D)], wsemS)
        if j + 1 < CPW:
            if j >= 1:
                wl[j - 1].wait()
                ws[j - 1].wait()
            gl[j + 1] = pltpu.async_copy(labt.at[idxL.at[j + 1]],
                                         bufL.at[(j + 1) % 2], gsemL)
            gs[j + 1] = pltpu.async_copy(subt.at[idxS.at[j + 1]],
                                         bufS.at[(j + 1) % 2], gsemS)
    wl[CPW - 2].wait()
    ws[CPW - 2].wait()
    wl[CPW - 1].wait()
    ws[CPW - 1].wait()


def kernel(lab_ids, subject_ids, lab_table, subject_table):
    labi = lab_ids.astype(jnp.int32).reshape(B // CHUNK, CHUNK)
    subi = subject_ids.astype(jnp.int32).reshape(B // CHUNK, CHUNK)
    mesh = plsc.VectorSubcoreMesh(core_axis_name="c", subcore_axis_name="s")
    f = pl.kernel(
        _body,
        mesh=mesh,
        out_type=jax.ShapeDtypeStruct((B, 2 * D), jnp.float32),
        scratch_types=[
            pltpu.VMEM((CPW, CHUNK), jnp.int32),
            pltpu.VMEM((CPW, CHUNK), jnp.int32),
            pltpu.VMEM((2, CHUNK, D), jnp.float32),
            pltpu.VMEM((2, CHUNK, D), jnp.float32),
            pltpu.SemaphoreType.DMA,
            pltpu.SemaphoreType.DMA,
            pltpu.SemaphoreType.DMA,
            pltpu.SemaphoreType.DMA,
        ],
    )
    return f(labi, subi, lab_table, subject_table)


# speedup vs baseline: 3.5465x; 2.9023x over previous
import jax
import jax.numpy as jnp
from jax import lax
from jax.experimental import pallas as pl
from jax.experimental.pallas import tpu as pltpu
from jax.experimental.pallas import tpu_sc as plsc

NC, NS = 2, 16
NW = NC * NS
B = 16384
D = 128
NL, NSUBJ = 30, 100
CHUNK = 128
CPW = B // (NW * CHUNK)


def _body(labi, subi, labt, subt, out, idxL, idxS, tabL, tabS, bufL, bufS,
          gsemL, gsemS, wsemL, wsemS):
    sid = lax.axis_index("s")
    wid = sid * NC + lax.axis_index("c")
    row0 = wid * CPW
    @pl.when(sid == 0)
    def _stage():
        pltpu.sync_copy(labt, tabL)
        pltpu.sync_copy(subt, tabS)
    pltpu.sync_copy(labi.at[pl.ds(row0, CPW)], idxL)
    pltpu.sync_copy(subi.at[pl.ds(row0, CPW)], idxS)
    plsc.subcore_barrier()
    gl = [None] * CPW
    gs = [None] * CPW
    wl = [None] * CPW
    ws = [None] * CPW
    gl[0] = pltpu.async_copy(tabL.at[idxL.at[0]], bufL.at[0], gsemL)
    gs[0] = pltpu.async_copy(tabS.at[idxS.at[0]], bufS.at[0], gsemS)
    for j in range(CPW):
        gl[j].wait()
        gs[j].wait()
        r = (row0 + j) * CHUNK
        wl[j] = pltpu.async_copy(bufL.at[j % 2],
                                 out.at[pl.ds(r, CHUNK), pl.ds(0, D)], wsemL)
        ws[j] = pltpu.async_copy(bufS.at[j % 2],
                                 out.at[pl.ds(r, CHUNK), pl.ds(D, D)], wsemS)
        if j + 1 < CPW:
            if j >= 1:
                wl[j - 1].wait()
                ws[j - 1].wait()
            gl[j + 1] = pltpu.async_copy(tabL.at[idxL.at[j + 1]],
                                         bufL.at[(j + 1) % 2], gsemL)
            gs[j + 1] = pltpu.async_copy(tabS.at[idxS.at[j + 1]],
                                         bufS.at[(j + 1) % 2], gsemS)
    wl[CPW - 2].wait()
    ws[CPW - 2].wait()
    wl[CPW - 1].wait()
    ws[CPW - 1].wait()


def kernel(lab_ids, subject_ids, lab_table, subject_table):
    labi = lab_ids.astype(jnp.int32).reshape(B // CHUNK, CHUNK)
    subi = subject_ids.astype(jnp.int32).reshape(B // CHUNK, CHUNK)
    mesh = plsc.VectorSubcoreMesh(core_axis_name="c", subcore_axis_name="s")
    f = pl.kernel(
        _body,
        mesh=mesh,
        out_type=jax.ShapeDtypeStruct((B, 2 * D), jnp.float32),
        scratch_types=[
            pltpu.VMEM((CPW, CHUNK), jnp.int32),
            pltpu.VMEM((CPW, CHUNK), jnp.int32),
            pltpu.VMEM_SHARED((NL, D), jnp.float32),
            pltpu.VMEM_SHARED((NSUBJ, D), jnp.float32),
            pltpu.VMEM((2, CHUNK, D), jnp.float32),
            pltpu.VMEM((2, CHUNK, D), jnp.float32),
            pltpu.SemaphoreType.DMA,
            pltpu.SemaphoreType.DMA,
            pltpu.SemaphoreType.DMA,
            pltpu.SemaphoreType.DMA,
        ],
    )
    return f(labi, subi, lab_table, subject_table)


# speedup vs baseline: 3.6719x; 1.0354x over previous
import jax
import jax.numpy as jnp
from jax import lax
from jax.experimental import pallas as pl
from jax.experimental.pallas import tpu as pltpu
from jax.experimental.pallas import tpu_sc as plsc

NC, NS = 2, 16
NW = NC * NS
B = 16384
D = 128
NL, NSUBJ = 30, 100
CHUNK = 128
CPW = B // (NW * CHUNK)


def _body(labi, subi, labt, subt, out, idxL, idxS, tabL, tabS, bufL, bufS,
          gsemL, gsemS, wsemL, wsemS):
    sid = lax.axis_index("s")
    wid = sid * NC + lax.axis_index("c")
    row0 = wid * CPW
    base = row0 * CHUNK
    @pl.when(sid == 0)
    def _stage():
        cL = pltpu.async_copy(labt, tabL, gsemL)
        cS = pltpu.async_copy(subt, tabS, gsemS)
        cL.wait()
        cS.wait()
    ic = []
    for j in range(CPW):
        ic.append(pltpu.async_copy(labi.at[pl.ds(base + j * CHUNK, CHUNK)],
                                   idxL.at[j], wsemL))
        ic.append(pltpu.async_copy(subi.at[pl.ds(base + j * CHUNK, CHUNK)],
                                   idxS.at[j], wsemS))
    for c in ic:
        c.wait()
    plsc.subcore_barrier()
    gl = [None] * CPW
    gs = [None] * CPW
    wl = [None] * CPW
    ws = [None] * CPW
    gl[0] = pltpu.async_copy(tabL.at[idxL.at[0]], bufL.at[0], gsemL)
    gs[0] = pltpu.async_copy(tabS.at[idxS.at[0]], bufS.at[0], gsemS)
    for j in range(CPW):
        gl[j].wait()
        gs[j].wait()
        r = (row0 + j) * CHUNK
        wl[j] = pltpu.async_copy(bufL.at[j % 2],
                                 out.at[pl.ds(r, CHUNK), pl.ds(0, D)], wsemL)
        ws[j] = pltpu.async_copy(bufS.at[j % 2],
                                 out.at[pl.ds(r, CHUNK), pl.ds(D, D)], wsemS)
        if j + 1 < CPW:
            if j >= 1:
                wl[j - 1].wait()
                ws[j - 1].wait()
            gl[j + 1] = pltpu.async_copy(tabL.at[idxL.at[j + 1]],
                                         bufL.at[(j + 1) % 2], gsemL)
            gs[j + 1] = pltpu.async_copy(tabS.at[idxS.at[j + 1]],
                                         bufS.at[(j + 1) % 2], gsemS)
    wl[CPW - 2].wait()
    ws[CPW - 2].wait()
    wl[CPW - 1].wait()
    ws[CPW - 1].wait()


def kernel(lab_ids, subject_ids, lab_table, subject_table):
    labi = lab_ids.astype(jnp.int32)
    subi = subject_ids.astype(jnp.int32)
    mesh = plsc.VectorSubcoreMesh(core_axis_name="c", subcore_axis_name="s")
    f = pl.kernel(
        _body,
        mesh=mesh,
        out_type=jax.ShapeDtypeStruct((B, 2 * D), jnp.float32),
        scratch_types=[
            pltpu.VMEM((CPW, CHUNK), jnp.int32),
            pltpu.VMEM((CPW, CHUNK), jnp.int32),
            pltpu.VMEM_SHARED((NL, D), jnp.float32),
            pltpu.VMEM_SHARED((NSUBJ, D), jnp.float32),
            pltpu.VMEM((2, CHUNK, D), jnp.float32),
            pltpu.VMEM((2, CHUNK, D), jnp.float32),
            pltpu.SemaphoreType.DMA,
            pltpu.SemaphoreType.DMA,
            pltpu.SemaphoreType.DMA,
            pltpu.SemaphoreType.DMA,
        ],
    )
    return f(labi, subi, lab_table, subject_table)
